# speedup vs baseline: 1.0454x; 1.0454x over previous
import functools
import math

import jax
import jax.numpy as jnp
from jax import lax
from jax.experimental import pallas as pl
from jax.experimental.pallas import tpu as pltpu
from jax.experimental.pallas import tpu_sc as plsc

_VOCAB = 1000000
_D = 128
_S = 200
_B = 1024
_N = _B * _S

_info = plsc.get_sparse_core_info()
_NC = _info.num_cores
_NS = _info.num_subcores
_NW = _NC * _NS
_PER_W = _N // _NW
_CHUNK = 128
_CHUNKS = _PER_W // _CHUNK


def _sc_gather(tok3d, table):
    mesh = plsc.VectorSubcoreMesh(core_axis_name="c", subcore_axis_name="s")

    @functools.partial(
        pl.kernel,
        out_type=jax.ShapeDtypeStruct((_N, _D), jnp.float32),
        mesh=mesh,
        scratch_types=[
            pltpu.VMEM((_CHUNKS, _CHUNK), jnp.int32),
            pltpu.VMEM((_CHUNK, _D), jnp.float32),
            pltpu.SemaphoreType.DMA,
        ],
    )
    def k(tok_hbm, table_hbm, out_hbm, idx_v, rows_v, gsem):
        wid = lax.axis_index("s") * _NC + lax.axis_index("c")
        base = wid * _PER_W
        pltpu.sync_copy(tok_hbm.at[wid], idx_v)

        def chunk(c, carry):
            pltpu.async_copy(table_hbm.at[idx_v.at[c]], rows_v, gsem).wait()
            pltpu.sync_copy(rows_v, out_hbm.at[pl.ds(base + c * _CHUNK, _CHUNK)])
            return carry

        lax.fori_loop(0, _CHUNKS, chunk, 0)

    return k(tok3d, table)


_BB = 8


def _tc_body(g_ref, pos_ref, w_ref, o_ref):
    scale = math.sqrt(_D)
    x = g_ref[...] + pos_ref[...][None]
    ws = w_ref[...] * scale
    xf = x.reshape(_BB * _S, _D)
    y = lax.dot_general(
        xf, ws, (((1,), (1,)), ((), ())), preferred_element_type=jnp.float32
    )
    o_ref[...] = y.reshape(_BB, _S, _D)


def _tc_project(g3d, pos, w):
    return pl.pallas_call(
        _tc_body,
        grid=(_B // _BB,),
        in_specs=[
            pl.BlockSpec((_BB, _S, _D), lambda i: (i, 0, 0)),
            pl.BlockSpec((_S, _D), lambda i: (0, 0)),
            pl.BlockSpec((_D, _D), lambda i: (0, 0)),
        ],
        out_specs=pl.BlockSpec((_BB, _S, _D), lambda i: (i, 0, 0)),
        out_shape=jax.ShapeDtypeStruct((_B, _S, _D), jnp.float32),
    )(g3d, pos, w)


@jax.jit
def kernel(tokens, emb_weight, pos_weight, proj_weight):
    tok3d = tokens.astype(jnp.int32).reshape(_NW, _CHUNKS, _CHUNK)
    gathered = _sc_gather(tok3d, emb_weight)
    g3d = gathered.reshape(_B, _S, _D)
    return _tc_project(g3d, pos_weight, proj_weight)


# speedup vs baseline: 1.2149x; 1.1621x over previous
import functools
import math

import jax
import jax.numpy as jnp
from jax import lax
from jax.experimental import pallas as pl
from jax.experimental.pallas import tpu as pltpu
from jax.experimental.pallas import tpu_sc as plsc

_VOCAB = 1000000
_D = 128
_S = 200
_B = 1024
_N = _B * _S

_info = plsc.get_sparse_core_info()
_NC = _info.num_cores
_NS = _info.num_subcores
_NW = _NC * _NS
_PER_W = _N // _NW
_CHUNK = 128
_CHUNKS = _PER_W // _CHUNK


def _sc_gather(tok3d, table):
    mesh = plsc.VectorSubcoreMesh(core_axis_name="c", subcore_axis_name="s")

    @functools.partial(
        pl.kernel,
        out_type=jax.ShapeDtypeStruct((_N, _D), jnp.float32),
        mesh=mesh,
        scratch_types=[
            pltpu.VMEM((_CHUNKS, _CHUNK), jnp.int32),
            pltpu.VMEM((_CHUNK, _D), jnp.float32),
            pltpu.VMEM((_CHUNK, _D), jnp.float32),
            pltpu.SemaphoreType.DMA,
            pltpu.SemaphoreType.DMA,
        ],
    )
    def k(tok_hbm, table_hbm, out_hbm, idx_v, rows0, rows1, sem0, sem1):
        wid = lax.axis_index("s") * _NC + lax.axis_index("c")
        base = wid * _PER_W
        pltpu.sync_copy(tok_hbm.at[wid], idx_v)

        pltpu.async_copy(table_hbm.at[idx_v.at[0]], rows0, sem0)

        def pair(i, carry):
            c0 = 2 * i
            pltpu.async_copy(table_hbm.at[idx_v.at[c0 + 1]], rows1, sem1)
            pltpu.make_async_copy(table_hbm.at[idx_v.at[c0]], rows0, sem0).wait()
            pltpu.sync_copy(rows0, out_hbm.at[pl.ds(base + c0 * _CHUNK, _CHUNK)])

            @pl.when(c0 + 2 < _CHUNKS)
            def _():
                pltpu.async_copy(table_hbm.at[idx_v.at[c0 + 2]], rows0, sem0)

            pltpu.make_async_copy(table_hbm.at[idx_v.at[c0 + 1]], rows1, sem1).wait()
            pltpu.sync_copy(
                rows1, out_hbm.at[pl.ds(base + (c0 + 1) * _CHUNK, _CHUNK)]
            )
            return carry

        lax.fori_loop(0, _CHUNKS // 2, pair, 0)

    return k(tok3d, table)


_BB = 8


def _tc_body(g_ref, pos_ref, w_ref, o_ref):
    scale = math.sqrt(_D)
    x = g_ref[...] + pos_ref[...][None]
    ws = w_ref[...] * scale
    xf = x.reshape(_BB * _S, _D)
    y = lax.dot_general(
        xf, ws, (((1,), (1,)), ((), ())), preferred_element_type=jnp.float32
    )
    o_ref[...] = y.reshape(_BB, _S, _D)


def _tc_project(g3d, pos, w):
    return pl.pallas_call(
        _tc_body,
        grid=(_B // _BB,),
        in_specs=[
            pl.BlockSpec((_BB, _S, _D), lambda i: (i, 0, 0)),
            pl.BlockSpec((_S, _D), lambda i: (0, 0)),
            pl.BlockSpec((_D, _D), lambda i: (0, 0)),
        ],
        out_specs=pl.BlockSpec((_BB, _S, _D), lambda i: (i, 0, 0)),
        out_shape=jax.ShapeDtypeStruct((_B, _S, _D), jnp.float32),
    )(g3d, pos, w)


@jax.jit
def kernel(tokens, emb_weight, pos_weight, proj_weight):
    tok3d = tokens.astype(jnp.int32).reshape(_NW, _CHUNKS, _CHUNK)
    gathered = _sc_gather(tok3d, emb_weight)
    g3d = gathered.reshape(_B, _S, _D)
    return _tc_project(g3d, pos_weight, proj_weight)
